# baseline (speedup 1.0000x reference)
import math

import jax
import jax.numpy as jnp
from jax import lax
from jax.experimental import pallas as pl
from jax.experimental.pallas import tpu as pltpu
from jax.experimental.pallas import tpu_sc as plsc

VOCAB_SIZE = 1000000
EMB_SIZE = 64
BATCH = 4096
SEQ_LEN = 200

NUM_CORES = 2
NUM_SUBCORES = 16
NUM_WORKERS = NUM_CORES * NUM_SUBCORES
B_PER_WORKER = BATCH // NUM_WORKERS
LANES = 16
SCALE = math.sqrt(EMB_SIZE)
NROWBUF = 4


def _body(tok_hbm, emb_hbm, pe_hbm, out_hbm,
          pe_v, idx_v, rows_v, stage_v0, stage_v1,
          gsem0, gsem1, gsem2, gsem3, osem0, osem1):
    stage_v = (stage_v0, stage_v1)
    cid = lax.axis_index("c")
    sid = lax.axis_index("s")
    wid = sid * NUM_CORES + cid
    bbase = wid * B_PER_WORKER

    gsem = (gsem0, gsem1, gsem2, gsem3)
    osem = (osem0, osem1)

    pltpu.sync_copy(pe_hbm, pe_v)
    pltpu.sync_copy(tok_hbm.at[:, pl.ds(bbase, B_PER_WORKER)], idx_v)

    iota = lax.iota(jnp.int32, LANES)
    idx_c = [(iota + (j * LANES)) * 128 for j in range(EMB_SIZE // LANES)]

    def fire(s, rb):
        pltpu.async_copy(emb_hbm.at[idx_v.at[s]], rows_v.at[rb], gsem[rb])

    def step(s, rb, ob, do_owait, do_fire):
        if do_fire:
            fire(s + 2, (rb + 2) % NROWBUF)
        pltpu.make_async_copy(emb_hbm.at[pl.ds(0, B_PER_WORKER)],
                              rows_v.at[rb], gsem[rb]).wait()
        if do_owait:
            pltpu.make_async_copy(stage_v[ob], out_hbm.at[0, :, 0, pl.ds(0, 8192 // 8)],
                                  osem[ob]).wait()

        pe_regs = [pe_v[s, pl.ds(j * LANES, LANES)]
                   for j in range(EMB_SIZE // LANES)]

        @pl.loop(0, B_PER_WORKER)
        def _b_loop(bb):
            bvec = jnp.full((LANES,), 0, jnp.int32) + bb
            for j in range(EMB_SIZE // LANES):
                vals = rows_v[rb, bb, pl.ds(j * LANES, LANES)] * SCALE + pe_regs[j]
                plsc.store_scatter(stage_v[ob], [idx_c[j] + bvec], vals)

        for ch in range(8):
            pltpu.async_copy(stage_v[ob].at[pl.ds(ch * 1024, 1024)],
                             out_hbm.at[s, ch, wid], osem[ob])

    fire(0, 0)
    fire(1, 1)
    step(0, 0, 0, do_owait=False, do_fire=True)
    step(1, 1, 1, do_owait=False, do_fire=True)

    @pl.loop(2, SEQ_LEN - 2, step=4)
    def _s_loop(c):
        step(c, 2, 0, do_owait=True, do_fire=True)
        step(c + 1, 3, 1, do_owait=True, do_fire=True)
        step(c + 2, 0, 0, do_owait=True, do_fire=True)
        step(c + 3, 1, 1, do_owait=True, do_fire=True)

    step(SEQ_LEN - 2, 2, 0, do_owait=True, do_fire=False)
    step(SEQ_LEN - 1, 3, 1, do_owait=True, do_fire=False)
    pltpu.make_async_copy(stage_v[0], out_hbm.at[0, :, 0, pl.ds(0, 8192 // 8)],
                          osem[0]).wait()
    pltpu.make_async_copy(stage_v[1], out_hbm.at[0, :, 0, pl.ds(0, 8192 // 8)],
                          osem[1]).wait()


def kernel(tokens, embedding, positional_embedding):
    pe = positional_embedding[0, :SEQ_LEN]
    tok_t = tokens.astype(jnp.int32).T

    run = pl.kernel(
        _body,
        out_type=jax.ShapeDtypeStruct((SEQ_LEN, 8, NUM_WORKERS, 8 * 128),
                                      jnp.float32),
        mesh=plsc.VectorSubcoreMesh(core_axis_name="c", subcore_axis_name="s"),
        compiler_params=pltpu.CompilerParams(use_tc_tiling_on_sc=False, needs_layout_passes=False),
        scratch_types=[
            pltpu.VMEM((SEQ_LEN, EMB_SIZE), jnp.float32),
            pltpu.VMEM((SEQ_LEN, B_PER_WORKER), jnp.int32),
            pltpu.VMEM((NROWBUF, B_PER_WORKER, EMB_SIZE), jnp.float32),
            pltpu.VMEM((8 * 8 * 128,), jnp.float32),
            pltpu.VMEM((8 * 8 * 128,), jnp.float32),
            pltpu.SemaphoreType.DMA,
            pltpu.SemaphoreType.DMA,
            pltpu.SemaphoreType.DMA,
            pltpu.SemaphoreType.DMA,
            pltpu.SemaphoreType.DMA,
            pltpu.SemaphoreType.DMA,
        ],
    )
    out = run(tok_t, embedding, pe)
    out = out.reshape(SEQ_LEN, 8, NUM_WORKERS, 8, 128)
    return out.transpose(2, 4, 0, 1, 3).reshape(BATCH, SEQ_LEN, EMB_SIZE)


# speedup vs baseline: 1.0068x; 1.0068x over previous
import math

import jax
import jax.numpy as jnp
from jax import lax
from jax.experimental import pallas as pl
from jax.experimental.pallas import tpu as pltpu
from jax.experimental.pallas import tpu_sc as plsc

VOCAB_SIZE = 1000000
EMB_SIZE = 64
BATCH = 4096
SEQ_LEN = 200

NUM_CORES = 2
NUM_SUBCORES = 16
NUM_WORKERS = NUM_CORES * NUM_SUBCORES
B_PER_WORKER = BATCH // NUM_WORKERS
LANES = 16
SCALE = math.sqrt(EMB_SIZE)
NROWBUF = 4


def _body(tok_hbm, emb_hbm, pe_hbm, out_hbm,
          pe_v, idx_v, rows_v, stage_v0, stage_v1,
          gsem0, gsem1, gsem2, gsem3, osem0, osem1):
    stage_v = (stage_v0, stage_v1)
    cid = lax.axis_index("c")
    sid = lax.axis_index("s")
    wid = sid * NUM_CORES + cid
    bbase = wid * B_PER_WORKER

    gsem = (gsem0, gsem1, gsem2, gsem3)
    osem = (osem0, osem1)

    pltpu.sync_copy(pe_hbm, pe_v)
    pltpu.sync_copy(tok_hbm.at[:, pl.ds(bbase, B_PER_WORKER)], idx_v)

    iota = lax.iota(jnp.int32, LANES)
    idx_c = [(iota + (j * LANES)) * 128 for j in range(EMB_SIZE // LANES)]

    def fire(s, rb):
        pltpu.async_copy(emb_hbm.at[idx_v.at[s]], rows_v.at[rb], gsem[rb])

    def step(s, rb, ob, do_owait, do_fire):
        if do_fire:
            fire(s + 2, (rb + 2) % NROWBUF)
        pltpu.make_async_copy(emb_hbm.at[pl.ds(0, B_PER_WORKER)],
                              rows_v.at[rb], gsem[rb]).wait()
        if do_owait:
            pltpu.make_async_copy(stage_v[ob], out_hbm.at[0, :, 0, pl.ds(0, 1024)],
                                  osem[ob]).wait()

        pe_regs = [pe_v[s, pl.ds(j * LANES, LANES)]
                   for j in range(EMB_SIZE // LANES)]

        stage_flat = stage_v[ob]

        @pl.loop(0, B_PER_WORKER, unroll=4)
        def _b_loop(bb):
            bvec = jnp.full((LANES,), 0, jnp.int32) + bb
            for j in range(EMB_SIZE // LANES):
                vals = rows_v[rb, bb, pl.ds(j * LANES, LANES)] * SCALE + pe_regs[j]
                plsc.store_scatter(stage_flat, [idx_c[j] + bvec], vals)

        for ch in range(8):
            pltpu.async_copy(stage_v[ob].at[pl.ds(ch * 1024, 1024)],
                             out_hbm.at[s, ch, wid], osem[ob])

    fire(0, 0)
    fire(1, 1)
    step(0, 0, 0, do_owait=False, do_fire=True)
    step(1, 1, 1, do_owait=False, do_fire=True)

    @pl.loop(2, SEQ_LEN - 2, step=4)
    def _s_loop(c):
        step(c, 2, 0, do_owait=True, do_fire=True)
        step(c + 1, 3, 1, do_owait=True, do_fire=True)
        step(c + 2, 0, 0, do_owait=True, do_fire=True)
        step(c + 3, 1, 1, do_owait=True, do_fire=True)

    step(SEQ_LEN - 2, 2, 0, do_owait=True, do_fire=False)
    step(SEQ_LEN - 1, 3, 1, do_owait=True, do_fire=False)
    pltpu.make_async_copy(stage_v[0], out_hbm.at[0, :, 0, pl.ds(0, 1024)],
                          osem[0]).wait()
    pltpu.make_async_copy(stage_v[1], out_hbm.at[0, :, 0, pl.ds(0, 1024)],
                          osem[1]).wait()


def kernel(tokens, embedding, positional_embedding):
    pe = positional_embedding[0, :SEQ_LEN]
    tok_t = tokens.astype(jnp.int32).T

    run = pl.kernel(
        _body,
        out_type=jax.ShapeDtypeStruct((SEQ_LEN, 8, NUM_WORKERS, 8 * 128),
                                      jnp.float32),
        mesh=plsc.VectorSubcoreMesh(core_axis_name="c", subcore_axis_name="s"),
        compiler_params=pltpu.CompilerParams(use_tc_tiling_on_sc=False, needs_layout_passes=False),
        scratch_types=[
            pltpu.VMEM((SEQ_LEN, EMB_SIZE), jnp.float32),
            pltpu.VMEM((SEQ_LEN, B_PER_WORKER), jnp.int32),
            pltpu.VMEM((NROWBUF, B_PER_WORKER, EMB_SIZE), jnp.float32),
            pltpu.VMEM((8 * 8 * 128,), jnp.float32),
            pltpu.VMEM((8 * 8 * 128,), jnp.float32),
            pltpu.SemaphoreType.DMA,
            pltpu.SemaphoreType.DMA,
            pltpu.SemaphoreType.DMA,
            pltpu.SemaphoreType.DMA,
            pltpu.SemaphoreType.DMA,
            pltpu.SemaphoreType.DMA,
        ],
    )
    out = run(tok_t, embedding, pe)
    out = out.reshape(SEQ_LEN, 8, NUM_WORKERS, 8, 128)
    return out.transpose(2, 4, 0, 1, 3).reshape(BATCH, SEQ_LEN, EMB_SIZE)


# speedup vs baseline: 1.4113x; 1.4017x over previous
import math

import jax
import jax.numpy as jnp
from jax import lax
from jax.experimental import pallas as pl
from jax.experimental.pallas import tpu as pltpu
from jax.experimental.pallas import tpu_sc as plsc

VOCAB_SIZE = 1000000
EMB_SIZE = 64
BATCH = 4096
SEQ_LEN = 200

NUM_CORES = 2
NUM_SUBCORES = 16
NUM_WORKERS = NUM_CORES * NUM_SUBCORES
SEQ_PER_WORKER = BATCH // NUM_WORKERS
LANES = 16
SCALE = math.sqrt(EMB_SIZE)


def _body(tok_hbm, emb_hbm, pe_hbm, out_hbm,
          pe_v, idx_v, rows_v, gsem0, gsem1, osem0, osem1):
    cid = lax.axis_index("c")
    sid = lax.axis_index("s")
    wid = sid * NUM_CORES + cid
    base = wid * SEQ_PER_WORKER

    gsem = (gsem0, gsem1)
    osem = (osem0, osem1)

    pltpu.sync_copy(pe_hbm, pe_v)
    pltpu.sync_copy(tok_hbm.at[pl.ds(base, SEQ_PER_WORKER)], idx_v)

    def fire(cc, b):
        pltpu.async_copy(emb_hbm.at[idx_v.at[cc]], rows_v.at[b], gsem[b])

    def step(cc, b, do_owait, do_fire):
        if do_owait:
            pltpu.make_async_copy(rows_v.at[1 - b], out_hbm.at[base],
                                  osem[1 - b]).wait()
        if do_fire:
            fire(cc + 1, 1 - b)
        pltpu.make_async_copy(emb_hbm.at[pl.ds(0, SEQ_LEN)], rows_v.at[b],
                              gsem[b]).wait()

        @pl.loop(0, SEQ_LEN)
        def _pos_loop(p):
            for j in range(EMB_SIZE // LANES):
                sl = pl.ds(j * LANES, LANES)
                rows_v[b, p, sl] = rows_v[b, p, sl] * SCALE + pe_v[p, sl]

        pltpu.async_copy(rows_v.at[b], out_hbm.at[base + cc], osem[b])

    fire(0, 0)
    step(0, 0, do_owait=False, do_fire=True)

    @pl.loop(1, SEQ_PER_WORKER - 1, step=2)
    def _seq_loop(c):
        step(c, 1, do_owait=True, do_fire=True)
        step(c + 1, 0, do_owait=True, do_fire=True)

    step(SEQ_PER_WORKER - 1, 1, do_owait=True, do_fire=False)
    pltpu.make_async_copy(rows_v.at[1], out_hbm.at[base], osem[1]).wait()


def kernel(tokens, embedding, positional_embedding):
    pe = positional_embedding[0, :SEQ_LEN]

    run = pl.kernel(
        _body,
        out_type=jax.ShapeDtypeStruct((BATCH, SEQ_LEN, EMB_SIZE), jnp.float32),
        mesh=plsc.VectorSubcoreMesh(core_axis_name="c", subcore_axis_name="s"),
        compiler_params=pltpu.CompilerParams(use_tc_tiling_on_sc=False),
        scratch_types=[
            pltpu.VMEM((SEQ_LEN, EMB_SIZE), jnp.float32),
            pltpu.VMEM((SEQ_PER_WORKER, SEQ_LEN), jnp.int32),
            pltpu.VMEM((2, SEQ_LEN, EMB_SIZE), jnp.float32),
            pltpu.SemaphoreType.DMA,
            pltpu.SemaphoreType.DMA,
            pltpu.SemaphoreType.DMA,
            pltpu.SemaphoreType.DMA,
        ],
    )
    return run(tokens.astype(jnp.int32), embedding, pe)
